# baseline (speedup 1.0000x reference)
import functools

import jax
import jax.numpy as jnp
from jax import lax
from jax.experimental import pallas as pl
from jax.experimental.pallas import tpu as pltpu
from jax.experimental.pallas import tpu_sc as plsc

_N = 10000
_E = 320000
_F_IN = 128
_H = 64
_C = 16

_NC = 2
_NS = 16
_NW = _NC * _NS
_K = 100
_NB = 10
_LD = 5

_EPT = _E // _NW
_NCH = _EPT // _K
_NP = 10240
_RPT = _NP // _NS

_SC_PARAMS = pltpu.CompilerParams(use_tc_tiling_on_sc=False)


def _sc_mesh():
    return plsc.VectorSubcoreMesh(core_axis_name="c", subcore_axis_name="s")


def _make_hist():
    @functools.partial(
        pl.kernel,
        mesh=_sc_mesh(),
        compiler_params=_SC_PARAMS,
        out_type=jax.ShapeDtypeStruct((_NC, _NP, _C), jnp.float32),
        scratch_types=[
            pltpu.VMEM((_NCH, _K), jnp.int32),
            pltpu.VMEM((_K, _C), jnp.float32),
            pltpu.VMEM_SHARED((_NP, _C), jnp.float32),
            pltpu.SemaphoreType.DMA((_NB,)),
        ],
    )
    def hist(ones_hbm, dst_hbm, zeros_hbm, out_hbm, dst_v, rows_v, acc, ssem):
        c = lax.axis_index("c")
        s = lax.axis_index("s")
        wid = s * _NC + c
        r0 = s * _RPT
        pltpu.sync_copy(zeros_hbm.at[pl.ds(r0, _RPT)], acc.at[pl.ds(r0, _RPT)])
        pltpu.sync_copy(dst_hbm.at[pl.ds(wid * _NCH, _NCH)], dst_v)
        pltpu.sync_copy(ones_hbm, rows_v)
        plsc.subcore_barrier()

        def scat_start(i, t):
            pltpu.async_copy(rows_v, acc.at[dst_v.at[i]], ssem.at[t], add=True)

        def scat_wait(i, t):
            pltpu.make_async_copy(rows_v, acc.at[dst_v.at[i]],
                                  ssem.at[t]).wait()

        def step(i0, carry):
            for t in range(_NB):
                i = i0 * _NB + t

                @pl.when(i >= _NB)
                def _():
                    scat_wait(i - _NB, t)

                scat_start(i, t)
            return carry

        lax.fori_loop(0, _NCH // _NB, step, 0)
        for t in range(_NB):
            scat_wait(_NCH - _NB + t, t)
        plsc.subcore_barrier()
        pltpu.sync_copy(acc.at[pl.ds(r0, _RPT)], out_hbm.at[c, pl.ds(r0, _RPT)])

    return hist


def _make_spmm(d):
    @functools.partial(
        pl.kernel,
        mesh=_sc_mesh(),
        compiler_params=_SC_PARAMS,
        out_type=jax.ShapeDtypeStruct((_NC, _NP, d), jnp.float32),
        scratch_types=[
            pltpu.VMEM((_NCH, _K), jnp.int32),
            pltpu.VMEM((_NCH, _K), jnp.int32),
            pltpu.VMEM((_NB, _K, d), jnp.float32),
            pltpu.VMEM_SHARED((_NP, d), jnp.float32),
            pltpu.SemaphoreType.DMA((_NB,)),
            pltpu.SemaphoreType.DMA((_NB,)),
        ],
    )
    def spmm(rows_hbm, src_hbm, dst_hbm, zeros_hbm, out_hbm,
             src_v, dst_v, rows_v, acc, gsem, ssem):
        c = lax.axis_index("c")
        s = lax.axis_index("s")
        wid = s * _NC + c
        r0 = s * _RPT
        pltpu.sync_copy(zeros_hbm.at[pl.ds(r0, _RPT)], acc.at[pl.ds(r0, _RPT)])
        pltpu.sync_copy(src_hbm.at[pl.ds(wid * _NCH, _NCH)], src_v)
        pltpu.sync_copy(dst_hbm.at[pl.ds(wid * _NCH, _NCH)], dst_v)
        plsc.subcore_barrier()

        def gath_start(i, t):
            pltpu.async_copy(rows_hbm.at[src_v.at[i]], rows_v.at[t],
                             gsem.at[t])

        def gath_wait(i, t):
            pltpu.make_async_copy(rows_hbm.at[src_v.at[i]], rows_v.at[t],
                                  gsem.at[t]).wait()

        def scat_start(i, t):
            pltpu.async_copy(rows_v.at[t], acc.at[dst_v.at[i]], ssem.at[t],
                             add=True)

        def scat_wait(i, t):
            pltpu.make_async_copy(rows_v.at[t], acc.at[dst_v.at[i]],
                                  ssem.at[t]).wait()

        for t in range(_LD):
            gath_start(t, t)

        def step(i0, carry):
            for t in range(_NB):
                i = i0 * _NB + t
                tg = (t + _LD) % _NB

                @pl.when(i + _LD < _NCH)
                def _():
                    @pl.when(i + _LD >= _NB)
                    def _():
                        scat_wait(i + _LD - _NB, tg)

                    gath_start(i + _LD, tg)

                gath_wait(i, t)
                scat_start(i, t)
            return carry

        lax.fori_loop(0, _NCH // _NB, step, 0)
        for t in range(_NB):
            scat_wait(_NCH - _NB + t, t)
        plsc.subcore_barrier()
        pltpu.sync_copy(acc.at[pl.ds(r0, _RPT)], out_hbm.at[c, pl.ds(r0, _RPT)])

    return spmm


_BT = 640
_GRID = _NP // _BT
_BH = _BT * _C // 128
_BP = _BT * _H // 128


def _dinv_packed(hp_ref):
    deg = hp_ref[0] + hp_ref[1]
    return lax.rsqrt(jnp.maximum(deg, 1.0))


def _iota2(shape, dim):
    return lax.broadcasted_iota(jnp.int32, shape, dim)


def _sel(cond):
    return jnp.where(cond, 1.0, 0.0).astype(jnp.float32)


def _dinv_replicated(hp_ref):
    dp = _dinv_packed(hp_ref)
    segs = [jnp.concatenate([dp[:, 16 * k:16 * (k + 1)]] * 8, axis=1)
            for k in range(8)]
    return jnp.stack(segs, axis=1).reshape(_BT, 128)


def _dot(a, b):
    return jnp.dot(a, b, preferred_element_type=jnp.float32,
                   precision=lax.Precision.HIGHEST)


def _pack_rows(v, w):
    g = 128 // w
    rows = _BT * w // 128
    wide = jnp.concatenate([v] * g, axis=1)
    m = _sel(_iota2((_BT, 128), 1) // w == _iota2((_BT, 128), 0) % g)
    perm = _sel(_iota2((rows, _BT), 1) // g == _iota2((rows, _BT), 0))
    return _dot(perm, wide * m)


def _tc_mm1(hpv, x, w1):
    def body(hp_ref, x_ref, w1_ref, o_ref):
        dinv = _dinv_replicated(hp_ref)
        h = _dot(x_ref[...], w1_ref[...])
        o_ref[...] = _pack_rows(h * dinv[:, :_H], _H)

    return pl.pallas_call(
        body,
        grid=(_GRID,),
        in_specs=[
            pl.BlockSpec((_NC, _BH, 128), lambda i: (0, i, 0)),
            pl.BlockSpec((_BT, _F_IN), lambda i: (i, 0)),
            pl.BlockSpec((_F_IN, _H), lambda i: (0, 0)),
        ],
        out_specs=pl.BlockSpec((_BP, 128), lambda i: (i, 0)),
        out_shape=jax.ShapeDtypeStruct((_NP * _H // 128, 128), jnp.float32),
    )(hpv, x, w1)


def _tc_mm2(hpv, pv, w2):
    def body(hp_ref, p_ref, w2_ref, o_ref):
        dinv = _dinv_replicated(hp_ref)
        ps = p_ref[0] + p_ref[1]
        even, odd = ps[:, :_H], ps[:, _H:]
        un = jnp.stack([even, odd], axis=1).reshape(_BT, _H)
        h2 = jnp.maximum(un * dinv[:, :_H], 0.0)
        o2 = _dot(h2, w2_ref[...]) * dinv[:, :_C]
        o_ref[...] = _pack_rows(o2, _C)

    return pl.pallas_call(
        body,
        grid=(_GRID,),
        in_specs=[
            pl.BlockSpec((_NC, _BH, 128), lambda i: (0, i, 0)),
            pl.BlockSpec((_NC, _BP, 128), lambda i: (0, i, 0)),
            pl.BlockSpec((_H, _C), lambda i: (0, 0)),
        ],
        out_specs=pl.BlockSpec((_BH, 128), lambda i: (i, 0)),
        out_shape=jax.ShapeDtypeStruct((_NP * _C // 128, 128), jnp.float32),
    )(hpv, pv, w2)


def _tc_out(hpv, qv):
    def body(hp_ref, q_ref, o_ref):
        o = (q_ref[0] + q_ref[1]) * _dinv_packed(hp_ref)
        p2 = _sel(_iota2((_BT, _BH), 1) == _iota2((_BT, _BH), 0) // 8)
        rows = _dot(p2, o)
        m = _sel(_iota2((_BT, 128), 1) // _C == _iota2((_BT, 128), 0) % 8)
        s = _sel(_iota2((128, _C), 0) % _C == _iota2((128, _C), 1))
        o_ref[...] = _dot(rows * m, s)

    return pl.pallas_call(
        body,
        grid=(_GRID,),
        in_specs=[
            pl.BlockSpec((_NC, _BH, 128), lambda i: (0, i, 0)),
            pl.BlockSpec((_NC, _BH, 128), lambda i: (0, i, 0)),
        ],
        out_specs=pl.BlockSpec((_BT, _C), lambda i: (i, 0)),
        out_shape=jax.ShapeDtypeStruct((_N, _C), jnp.float32),
    )(hpv, qv)


def kernel(x, edge_index, W1, W2):
    src = edge_index[0].reshape(_E // _K, _K)
    dst = edge_index[1].reshape(_E // _K, _K)
    ones_k = jnp.ones((_K, _C), jnp.float32)
    zeros16 = jnp.zeros((_NP, _C), jnp.float32)
    zeros64 = jnp.zeros((_NP, _H), jnp.float32)

    hp = _make_hist()(ones_k, dst, zeros16)
    hpv = hp.reshape(_NC, _NP * _C // 128, 128)
    h1p = _tc_mm1(hpv, x, W1)
    h1v = h1p.reshape(_NP, _H)
    p = _make_spmm(_H)(h1v, src, dst, zeros64)
    pv = p.reshape(_NC, _NP * _H // 128, 128)
    o2p = _tc_mm2(hpv, pv, W2)
    o2v = o2p.reshape(_NP, _C)
    q = _make_spmm(_C)(o2v, src, dst, zeros16)
    qv = q.reshape(_NC, _NP * _C // 128, 128)
    return _tc_out(hpv, qv)


# speedup vs baseline: 1.1123x; 1.1123x over previous
import functools

import jax
import jax.numpy as jnp
from jax import lax
from jax.experimental import pallas as pl
from jax.experimental.pallas import tpu as pltpu
from jax.experimental.pallas import tpu_sc as plsc

_N = 10000
_E = 320000
_F_IN = 128
_H = 64
_C = 16

_NC = 2
_NS = 16
_NW = _NC * _NS
_K = 100
_NB = 10
_LD = 5

_EPT = _E // _NW
_NCH = _EPT // _K
_NP = 10240
_RPT = _NP // _NS

_SC_PARAMS = pltpu.CompilerParams(use_tc_tiling_on_sc=False)


def _sc_mesh():
    return plsc.VectorSubcoreMesh(core_axis_name="c", subcore_axis_name="s")


def _make_hist():
    @functools.partial(
        pl.kernel,
        mesh=_sc_mesh(),
        compiler_params=_SC_PARAMS,
        out_type=jax.ShapeDtypeStruct((_NC, _NP, _C), jnp.float32),
        scratch_types=[
            pltpu.VMEM((_NCH, _K), jnp.int32),
            pltpu.VMEM((_K, _C), jnp.float32),
            pltpu.VMEM_SHARED((_NP, _C), jnp.float32),
            pltpu.SemaphoreType.DMA((_NB,)),
        ],
    )
    def hist(ones_hbm, dst_hbm, zeros_hbm, out_hbm, dst_v, rows_v, acc, ssem):
        c = lax.axis_index("c")
        s = lax.axis_index("s")
        wid = s * _NC + c
        r0 = s * _RPT
        pltpu.sync_copy(zeros_hbm.at[pl.ds(r0, _RPT)], acc.at[pl.ds(r0, _RPT)])
        pltpu.sync_copy(dst_hbm.at[pl.ds(wid * _NCH, _NCH)], dst_v)
        pltpu.sync_copy(ones_hbm, rows_v)
        plsc.subcore_barrier()

        def scat_start(i, t):
            pltpu.async_copy(rows_v, acc.at[dst_v.at[i]], ssem.at[t], add=True)

        def scat_wait(i, t):
            pltpu.make_async_copy(rows_v, acc.at[dst_v.at[i]],
                                  ssem.at[t]).wait()

        def step(i0, carry):
            for t in range(_NB):
                i = i0 * _NB + t

                @pl.when(i >= _NB)
                def _():
                    scat_wait(i - _NB, t)

                scat_start(i, t)
            return carry

        lax.fori_loop(0, _NCH // _NB, step, 0)
        for t in range(_NB):
            scat_wait(_NCH - _NB + t, t)
        plsc.subcore_barrier()
        pltpu.sync_copy(acc.at[pl.ds(r0, _RPT)], out_hbm.at[c, pl.ds(r0, _RPT)])

    return hist


def _make_spmm(d):
    @functools.partial(
        pl.kernel,
        mesh=_sc_mesh(),
        compiler_params=_SC_PARAMS,
        out_type=jax.ShapeDtypeStruct((_NC, _NP, d), jnp.float32),
        scratch_types=[
            pltpu.VMEM((_NCH, _K), jnp.int32),
            pltpu.VMEM((_NCH, _K), jnp.int32),
            pltpu.VMEM((_NB, _K, d), jnp.float32),
            pltpu.VMEM_SHARED((_NP, d), jnp.float32),
            pltpu.SemaphoreType.DMA((_NB,)),
            pltpu.SemaphoreType.DMA((_NB,)),
        ],
    )
    def spmm(rows_hbm, src_hbm, dst_hbm, zeros_hbm, out_hbm,
             src_v, dst_v, rows_v, acc, gsem, ssem):
        c = lax.axis_index("c")
        s = lax.axis_index("s")
        wid = s * _NC + c
        r0 = s * _RPT
        pltpu.sync_copy(zeros_hbm.at[pl.ds(r0, _RPT)], acc.at[pl.ds(r0, _RPT)])
        pltpu.sync_copy(src_hbm.at[pl.ds(wid * _NCH, _NCH)], src_v)
        pltpu.sync_copy(dst_hbm.at[pl.ds(wid * _NCH, _NCH)], dst_v)
        plsc.subcore_barrier()

        def gath_start(i, t):
            pltpu.async_copy(rows_hbm.at[src_v.at[i]], rows_v.at[t],
                             gsem.at[t])

        def gath_wait(i, t):
            pltpu.make_async_copy(rows_hbm.at[src_v.at[i]], rows_v.at[t],
                                  gsem.at[t]).wait()

        def scat_start(i, t):
            pltpu.async_copy(rows_v.at[t], acc.at[dst_v.at[i]], ssem.at[t],
                             add=True)

        def scat_wait(i, t):
            pltpu.make_async_copy(rows_v.at[t], acc.at[dst_v.at[i]],
                                  ssem.at[t]).wait()

        for t in range(_LD):
            gath_start(t, t)

        def step(i0, carry):
            for t in range(_NB):
                i = i0 * _NB + t
                tg = (t + _LD) % _NB

                @pl.when(i + _LD < _NCH)
                def _():
                    @pl.when(i + _LD >= _NB)
                    def _():
                        scat_wait(i + _LD - _NB, tg)

                    gath_start(i + _LD, tg)

                gath_wait(i, t)
                scat_start(i, t)
            return carry

        lax.fori_loop(0, _NCH // _NB, step, 0)
        for t in range(_NB):
            scat_wait(_NCH - _NB + t, t)
        plsc.subcore_barrier()
        pltpu.sync_copy(acc.at[pl.ds(r0, _RPT)], out_hbm.at[c, pl.ds(r0, _RPT)])

    return spmm


_BT = 640
_GRID = _NP // _BT
_BH = _BT * _C // 128
_BP = _BT * _H // 128


def _dinv_packed(hp_ref):
    deg = hp_ref[0] + hp_ref[1]
    return lax.rsqrt(jnp.maximum(deg, 1.0))


def _dinv_replicated(hp_ref):
    dp = _dinv_packed(hp_ref)
    segs = [jnp.concatenate([dp[:, 16 * k:16 * (k + 1)]] * 8, axis=1)
            for k in range(8)]
    return jnp.stack(segs, axis=1).reshape(_BT, 128)


def _dot(a, b):
    return jnp.dot(a, b, preferred_element_type=jnp.float32)


def _pack_cols(v, w):
    g = 128 // w
    rows = _BT // g
    return jnp.concatenate([v[rows * k:rows * (k + 1)] for k in range(g)],
                           axis=1)


def _permute_idx(n, w):
    g = 128 // w
    rows = _BT // g
    q, u = n // _BT, n % _BT
    return q * _BT + g * (u % rows) + u // rows


def _tc_mm1(hpv, x, w1):
    def body(hp_ref, x_ref, w1_ref, o_ref):
        dinv = _dinv_replicated(hp_ref)
        h = _dot(x_ref[...], w1_ref[...])
        o_ref[...] = _pack_cols(h * dinv[:, :_H], _H)

    return pl.pallas_call(
        body,
        grid=(_GRID,),
        in_specs=[
            pl.BlockSpec((_NC, _BH, 128), lambda i: (0, i, 0)),
            pl.BlockSpec((_BT, _F_IN), lambda i: (i, 0)),
            pl.BlockSpec((_F_IN, _H), lambda i: (0, 0)),
        ],
        out_specs=pl.BlockSpec((_BP, 128), lambda i: (i, 0)),
        out_shape=jax.ShapeDtypeStruct((_NP * _H // 128, 128), jnp.float32),
    )(hpv, x, w1)


def _tc_mm2(hpv, pv, w2):
    def body(hp_ref, p_ref, w2_ref, o_ref):
        dinv = _dinv_replicated(hp_ref)
        ps = p_ref[0] + p_ref[1]
        even, odd = ps[:, :_H], ps[:, _H:]
        un = jnp.stack([even, odd], axis=1).reshape(_BT, _H)
        h2 = jnp.maximum(un * dinv[:, :_H], 0.0)
        o2 = _dot(h2, w2_ref[...]) * dinv[:, :_C]
        o_ref[...] = _pack_cols(o2, _C)

    return pl.pallas_call(
        body,
        grid=(_GRID,),
        in_specs=[
            pl.BlockSpec((_NC, _BH, 128), lambda i: (0, i, 0)),
            pl.BlockSpec((_NC, _BP, 128), lambda i: (0, i, 0)),
            pl.BlockSpec((_H, _C), lambda i: (0, 0)),
        ],
        out_specs=pl.BlockSpec((_BH, 128), lambda i: (i, 0)),
        out_shape=jax.ShapeDtypeStruct((_NP * _C // 128, 128), jnp.float32),
    )(hpv, pv, w2)


def _tc_out(hpv, qv):
    def body(hp_ref, q_ref, o_ref):
        o = (q_ref[0] + q_ref[1]) * _dinv_packed(hp_ref)
        segs = [o[:, _C * k:_C * (k + 1)] for k in range(8)]
        o_ref[...] = jnp.stack(segs, axis=1).reshape(_BT, _C)

    return pl.pallas_call(
        body,
        grid=(_GRID,),
        in_specs=[
            pl.BlockSpec((_NC, _BH, 128), lambda i: (0, i, 0)),
            pl.BlockSpec((_NC, _BH, 128), lambda i: (0, i, 0)),
        ],
        out_specs=pl.BlockSpec((_BT, _C), lambda i: (i, 0)),
        out_shape=jax.ShapeDtypeStruct((_N, _C), jnp.float32),
    )(hpv, qv)


def kernel(x, edge_index, W1, W2):
    src = edge_index[0]
    src1 = _permute_idx(src, _H).reshape(_E // _K, _K)
    src2 = _permute_idx(src, _C).reshape(_E // _K, _K)
    dst = edge_index[1].reshape(_E // _K, _K)
    ones_k = jnp.ones((_K, _C), jnp.float32)
    zeros16 = jnp.zeros((_NP, _C), jnp.float32)
    zeros64 = jnp.zeros((_NP, _H), jnp.float32)

    hp = _make_hist()(ones_k, dst, zeros16)
    hpv = hp.reshape(_NC, _NP * _C // 128, 128)
    h1p = _tc_mm1(hpv, x, W1)
    h1v = h1p.reshape(_NP, _H)
    p = _make_spmm(_H)(h1v, src1, dst, zeros64)
    pv = p.reshape(_NC, _NP * _H // 128, 128)
    o2p = _tc_mm2(hpv, pv, W2)
    o2v = o2p.reshape(_NP, _C)
    q = _make_spmm(_C)(o2v, src2, dst, zeros16)
    qv = q.reshape(_NC, _NP * _C // 128, 128)
    return _tc_out(hpv, qv)


# speedup vs baseline: 1.1426x; 1.0273x over previous
import functools

import jax
import jax.numpy as jnp
from jax import lax
from jax.experimental import pallas as pl
from jax.experimental.pallas import tpu as pltpu
from jax.experimental.pallas import tpu_sc as plsc

_N = 10000
_E = 320000
_F_IN = 128
_H = 64
_C = 16

_NC = 2
_NS = 16
_NW = _NC * _NS
_K = 100
_NB = 10
_LD = 5

_EPT = _E // _NW
_NCH = _EPT // _K
_NP = 10240
_RPT = _NP // _NS

_SC_PARAMS = pltpu.CompilerParams(use_tc_tiling_on_sc=False)


def _sc_mesh():
    return plsc.VectorSubcoreMesh(core_axis_name="c", subcore_axis_name="s")


def _make_hist():
    @functools.partial(
        pl.kernel,
        mesh=_sc_mesh(),
        compiler_params=_SC_PARAMS,
        out_type=jax.ShapeDtypeStruct((_NC, _NP, _C), jnp.float32),
        scratch_types=[
            pltpu.VMEM((_NCH, _K), jnp.int32),
            pltpu.VMEM((_K, _C), jnp.float32),
            pltpu.VMEM_SHARED((_NP, _C), jnp.float32),
            pltpu.SemaphoreType.DMA((_NB,)),
        ],
    )
    def hist(ones_hbm, dst_hbm, zeros_hbm, out_hbm, dst_v, rows_v, acc, ssem):
        c = lax.axis_index("c")
        s = lax.axis_index("s")
        wid = s * _NC + c
        r0 = s * _RPT
        pltpu.sync_copy(zeros_hbm.at[pl.ds(r0, _RPT)], acc.at[pl.ds(r0, _RPT)])
        pltpu.sync_copy(dst_hbm.at[pl.ds(wid * _NCH, _NCH)], dst_v)
        pltpu.sync_copy(ones_hbm, rows_v)
        plsc.subcore_barrier()

        def scat_start(i, t):
            pltpu.async_copy(rows_v, acc.at[dst_v.at[i]], ssem.at[t], add=True)

        def scat_wait(i, t):
            pltpu.make_async_copy(rows_v, acc.at[dst_v.at[i]],
                                  ssem.at[t]).wait()

        def step(i0, carry):
            for t in range(_NB):
                i = i0 * _NB + t

                @pl.when(i >= _NB)
                def _():
                    scat_wait(i - _NB, t)

                scat_start(i, t)
            return carry

        lax.fori_loop(0, _NCH // _NB, step, 0)
        for t in range(_NB):
            scat_wait(_NCH - _NB + t, t)
        plsc.subcore_barrier()
        pltpu.sync_copy(acc.at[pl.ds(r0, _RPT)], out_hbm.at[c, pl.ds(r0, _RPT)])

    return hist


def _make_spmm(d):
    @functools.partial(
        pl.kernel,
        mesh=_sc_mesh(),
        compiler_params=_SC_PARAMS,
        out_type=jax.ShapeDtypeStruct((_NC, _NP, d), jnp.float32),
        scratch_types=[
            pltpu.VMEM((_NCH, _K), jnp.int32),
            pltpu.VMEM((_NCH, _K), jnp.int32),
            pltpu.VMEM((_NB, _K, d), jnp.float32),
            pltpu.VMEM_SHARED((_NP, d), jnp.float32),
            pltpu.SemaphoreType.DMA((_NB,)),
            pltpu.SemaphoreType.DMA((_NB,)),
        ],
    )
    def spmm(rows_hbm, src_hbm, dst_hbm, zeros_hbm, out_hbm,
             src_v, dst_v, rows_v, acc, gsem, ssem):
        c = lax.axis_index("c")
        s = lax.axis_index("s")
        wid = s * _NC + c
        r0 = s * _RPT
        pltpu.sync_copy(zeros_hbm.at[pl.ds(r0, _RPT)], acc.at[pl.ds(r0, _RPT)])
        pltpu.sync_copy(src_hbm.at[pl.ds(wid * _NCH, _NCH)], src_v)
        pltpu.sync_copy(dst_hbm.at[pl.ds(wid * _NCH, _NCH)], dst_v)
        plsc.subcore_barrier()

        def gath_start(i, t):
            pltpu.async_copy(rows_hbm.at[src_v.at[i]], rows_v.at[t],
                             gsem.at[t])

        def gath_wait(i, t):
            pltpu.make_async_copy(rows_hbm.at[src_v.at[i]], rows_v.at[t],
                                  gsem.at[t]).wait()

        def scat_start(i, t):
            pltpu.async_copy(rows_v.at[t], acc.at[dst_v.at[i]], ssem.at[t],
                             add=True)

        def scat_wait(i, t):
            pltpu.make_async_copy(rows_v.at[t], acc.at[dst_v.at[i]],
                                  ssem.at[t]).wait()

        for t in range(_LD):
            gath_start(t, t)

        def step(i0, carry):
            for t in range(_NB):
                i = i0 * _NB + t
                tg = (t + _LD) % _NB

                @pl.when(i + _LD < _NCH)
                def _():
                    @pl.when(i + _LD >= _NB)
                    def _():
                        scat_wait(i + _LD - _NB, tg)

                    gath_start(i + _LD, tg)

                gath_wait(i, t)
                scat_start(i, t)
            return carry

        lax.fori_loop(0, _NCH // _NB, step, 0)
        for t in range(_NB):
            scat_wait(_NCH - _NB + t, t)
        plsc.subcore_barrier()
        pltpu.sync_copy(acc.at[pl.ds(r0, _RPT)], out_hbm.at[c, pl.ds(r0, _RPT)])

    return spmm


_BT = 512
_GRID = _NP // _BT
_BH = _BT * _C // 128
_BP = _BT * _H // 128


def _dinv_packed(hp_ref):
    deg = hp_ref[0] + hp_ref[1]
    return lax.rsqrt(jnp.maximum(deg, 1.0))


def _unpack16(v):
    segs = [v[:, _C * k:_C * (k + 1)] for k in range(8)]
    return jnp.stack(segs, axis=1).reshape(_BT, _C)


def _dinv_col(hp_ref):
    return _unpack16(_dinv_packed(hp_ref))[:, :1]


def _dot(a, b):
    return jnp.dot(a, b, preferred_element_type=jnp.float32)


def _pack_cols(v, w):
    g = 128 // w
    rows = _BT // g
    return jnp.concatenate([v[rows * k:rows * (k + 1)] for k in range(g)],
                           axis=1)


def _permute_idx(n, w):
    g = 128 // w
    rows = _BT // g
    q, u = n // _BT, n % _BT
    return q * _BT + g * (u % rows) + u // rows


def _tc_mm1(hpv, x, w1):
    def body(hp_ref, x_ref, w1_ref, o_ref):
        dinv = _dinv_col(hp_ref)
        h = _dot(x_ref[...], w1_ref[...])
        o_ref[...] = _pack_cols(h * dinv, _H)

    return pl.pallas_call(
        body,
        grid=(_GRID,),
        in_specs=[
            pl.BlockSpec((_NC, _BH, 128), lambda i: (0, i, 0)),
            pl.BlockSpec((_BT, _F_IN), lambda i: (i, 0)),
            pl.BlockSpec((_F_IN, _H), lambda i: (0, 0)),
        ],
        out_specs=pl.BlockSpec((_BP, 128), lambda i: (i, 0)),
        out_shape=jax.ShapeDtypeStruct((_NP * _H // 128, 128), jnp.float32),
    )(hpv, x, w1)


def _tc_mm2(hpv, pv, w2):
    def body(hp_ref, p_ref, w2_ref, o_ref):
        dinv = _dinv_col(hp_ref)
        ps = p_ref[0] + p_ref[1]
        even, odd = ps[:, :_H], ps[:, _H:]
        un = jnp.stack([even, odd], axis=1).reshape(_BT, _H)
        h2 = jnp.maximum(un * dinv, 0.0)
        o2 = _dot(h2, w2_ref[...]) * dinv
        o_ref[...] = _pack_cols(o2, _C)

    return pl.pallas_call(
        body,
        grid=(_GRID,),
        in_specs=[
            pl.BlockSpec((_NC, _BH, 128), lambda i: (0, i, 0)),
            pl.BlockSpec((_NC, _BP, 128), lambda i: (0, i, 0)),
            pl.BlockSpec((_H, _C), lambda i: (0, 0)),
        ],
        out_specs=pl.BlockSpec((_BH, 128), lambda i: (i, 0)),
        out_shape=jax.ShapeDtypeStruct((_NP * _C // 128, 128), jnp.float32),
    )(hpv, pv, w2)


def _tc_out(hpv, qv):
    def body(hp_ref, q_ref, o_ref):
        o_ref[...] = (q_ref[0] + q_ref[1]) * _dinv_packed(hp_ref)

    return pl.pallas_call(
        body,
        grid=(_GRID,),
        in_specs=[
            pl.BlockSpec((_NC, _BH, 128), lambda i: (0, i, 0)),
            pl.BlockSpec((_NC, _BH, 128), lambda i: (0, i, 0)),
        ],
        out_specs=pl.BlockSpec((_BH, 128), lambda i: (i, 0)),
        out_shape=jax.ShapeDtypeStruct((_NP * _C // 128, 128), jnp.float32),
    )(hpv, qv)


def kernel(x, edge_index, W1, W2):
    src = edge_index[0]
    src1 = _permute_idx(src, _H).reshape(_E // _K, _K)
    src2 = _permute_idx(src, _C).reshape(_E // _K, _K)
    dst = edge_index[1].reshape(_E // _K, _K)
    ones_k = jnp.ones((_K, _C), jnp.float32)
    zeros16 = jnp.zeros((_NP, _C), jnp.float32)
    zeros64 = jnp.zeros((_NP, _H), jnp.float32)

    hp = _make_hist()(ones_k, dst, zeros16)
    hpv = hp.reshape(_NC, _NP * _C // 128, 128)
    h1p = _tc_mm1(hpv, x, W1)
    h1v = h1p.reshape(_NP, _H)
    p = _make_spmm(_H)(h1v, src1, dst, zeros64)
    pv = p.reshape(_NC, _NP * _H // 128, 128)
    o2p = _tc_mm2(hpv, pv, W2)
    o2v = o2p.reshape(_NP, _C)
    q = _make_spmm(_C)(o2v, src2, dst, zeros16)
    qv = q.reshape(_NC, _NP * _C // 128, 128)
    return _tc_out(hpv, qv).reshape(_NP, _C)[:_N]


# speedup vs baseline: 1.1995x; 1.0498x over previous
import functools

import jax
import jax.numpy as jnp
from jax import lax
from jax.experimental import pallas as pl
from jax.experimental.pallas import tpu as pltpu
from jax.experimental.pallas import tpu_sc as plsc

_N = 10000
_E = 320000
_F_IN = 128
_H = 64
_C = 16

_NC = 2
_NS = 16
_NW = _NC * _NS
_K = 100
_NB = 10
_LD = 5

_EPT = _E // _NW
_NCH = _EPT // _K
_NP = 10240
_RPT = _NP // _NS

_SC_PARAMS = pltpu.CompilerParams(use_tc_tiling_on_sc=False)


def _sc_mesh():
    return plsc.VectorSubcoreMesh(core_axis_name="c", subcore_axis_name="s")


def _make_hist():
    @functools.partial(
        pl.kernel,
        mesh=_sc_mesh(),
        compiler_params=_SC_PARAMS,
        out_type=jax.ShapeDtypeStruct((_NC, _NP, _C), jnp.float32),
        scratch_types=[
            pltpu.VMEM((_NCH, _K), jnp.int32),
            pltpu.VMEM((_K, _C), jnp.float32),
            pltpu.VMEM_SHARED((_NP, _C), jnp.float32),
            pltpu.SemaphoreType.DMA((_NB,)),
        ],
    )
    def hist(ones_hbm, dst_hbm, zeros_hbm, out_hbm, dst_v, rows_v, acc, ssem):
        c = lax.axis_index("c")
        s = lax.axis_index("s")
        wid = s * _NC + c
        r0 = s * _RPT
        pltpu.sync_copy(zeros_hbm.at[pl.ds(r0, _RPT)], acc.at[pl.ds(r0, _RPT)])
        pltpu.sync_copy(dst_hbm.at[pl.ds(wid * _NCH, _NCH)], dst_v)
        pltpu.sync_copy(ones_hbm, rows_v)
        plsc.subcore_barrier()

        def scat_start(i, t):
            pltpu.async_copy(rows_v, acc.at[dst_v.at[i]], ssem.at[t], add=True)

        def scat_wait(i, t):
            pltpu.make_async_copy(rows_v, acc.at[dst_v.at[i]],
                                  ssem.at[t]).wait()

        def step(i0, carry):
            for t in range(_NB):
                i = i0 * _NB + t

                @pl.when(i >= _NB)
                def _():
                    scat_wait(i - _NB, t)

                scat_start(i, t)
            return carry

        lax.fori_loop(0, _NCH // _NB, step, 0)
        for t in range(_NB):
            scat_wait(_NCH - _NB + t, t)
        plsc.subcore_barrier()
        pltpu.sync_copy(acc.at[pl.ds(r0, _RPT)], out_hbm.at[c, pl.ds(r0, _RPT)])

    return hist


def _make_spmm(d):
    @functools.partial(
        pl.kernel,
        mesh=_sc_mesh(),
        compiler_params=_SC_PARAMS,
        out_type=jax.ShapeDtypeStruct((_NC, _NP, d), jnp.float32),
        scratch_types=[
            pltpu.VMEM((_NCH, _K), jnp.int32),
            pltpu.VMEM((_NCH, _K), jnp.int32),
            pltpu.VMEM((_NB, _K, d), jnp.float32),
            pltpu.VMEM_SHARED((_NP, d), jnp.float32),
            pltpu.SemaphoreType.DMA((_NB,)),
            pltpu.SemaphoreType.DMA((_NB,)),
        ],
    )
    def spmm(rows_hbm, src_hbm, dst_hbm, zeros_hbm, out_hbm,
             src_v, dst_v, rows_v, acc, gsem, ssem):
        c = lax.axis_index("c")
        s = lax.axis_index("s")
        wid = s * _NC + c
        r0 = s * _RPT
        pltpu.sync_copy(zeros_hbm.at[pl.ds(r0, _RPT)], acc.at[pl.ds(r0, _RPT)])
        pltpu.sync_copy(src_hbm.at[pl.ds(wid * _NCH, _NCH)], src_v)
        pltpu.sync_copy(dst_hbm.at[pl.ds(wid * _NCH, _NCH)], dst_v)
        plsc.subcore_barrier()

        def gath_start(i, t):
            pltpu.async_copy(rows_hbm.at[src_v.at[i]], rows_v.at[t],
                             gsem.at[t])

        def gath_wait(i, t):
            pltpu.make_async_copy(rows_hbm.at[src_v.at[i]], rows_v.at[t],
                                  gsem.at[t]).wait()

        def scat_start(i, t):
            pltpu.async_copy(rows_v.at[t], acc.at[dst_v.at[i]], ssem.at[t],
                             add=True)

        def scat_wait(i, t):
            pltpu.make_async_copy(rows_v.at[t], acc.at[dst_v.at[i]],
                                  ssem.at[t]).wait()

        for t in range(_LD):
            gath_start(t, t)

        def step(i0, carry):
            for t in range(_NB):
                i = i0 * _NB + t
                tg = (t + _LD) % _NB

                @pl.when(i + _LD < _NCH)
                def _():
                    @pl.when(i + _LD >= _NB)
                    def _():
                        scat_wait(i + _LD - _NB, tg)

                    gath_start(i + _LD, tg)

                gath_wait(i, t)
                scat_start(i, t)
            return carry

        lax.fori_loop(0, _NCH // _NB, step, 0)
        for t in range(_NB):
            scat_wait(_NCH - _NB + t, t)
        plsc.subcore_barrier()
        pltpu.sync_copy(acc.at[pl.ds(r0, _RPT)], out_hbm.at[c, pl.ds(r0, _RPT)])

    return spmm


_BT = 1024
_GRID = _NP // _BT
_BH = _BT * _C // 128
_BP = _BT * _H // 128


def _dinv_packed(hp_ref):
    deg = hp_ref[0] + hp_ref[1]
    return lax.rsqrt(jnp.maximum(deg, 1.0))


def _unpack16(v):
    segs = [v[:, _C * k:_C * (k + 1)] for k in range(8)]
    return jnp.stack(segs, axis=1).reshape(_BT, _C)


def _dinv_col(hp_ref):
    return _unpack16(_dinv_packed(hp_ref))[:, :1]


def _dot(a, b):
    return jnp.dot(a, b, preferred_element_type=jnp.float32)


def _pack_cols(v, w):
    g = 128 // w
    rows = _BT // g
    return jnp.concatenate([v[rows * k:rows * (k + 1)] for k in range(g)],
                           axis=1)


def _permute_idx(n, w):
    g = 128 // w
    rows = _BT // g
    q, u = n // _BT, n % _BT
    return q * _BT + g * (u % rows) + u // rows


def _tc_mm1(hpv, x, w1):
    def body(hp_ref, x_ref, w1_ref, o_ref):
        dinv = _dinv_col(hp_ref)
        h = _dot(x_ref[...], w1_ref[...])
        o_ref[...] = _pack_cols(h * dinv, _H)

    return pl.pallas_call(
        body,
        grid=(_GRID,),
        in_specs=[
            pl.BlockSpec((_NC, _BH, 128), lambda i: (0, i, 0)),
            pl.BlockSpec((_BT, _F_IN), lambda i: (i, 0)),
            pl.BlockSpec((_F_IN, _H), lambda i: (0, 0)),
        ],
        out_specs=pl.BlockSpec((_BP, 128), lambda i: (i, 0)),
        out_shape=jax.ShapeDtypeStruct((_NP * _H // 128, 128), jnp.float32),
    )(hpv, x, w1)


def _tc_mm2(hpv, pv, w2):
    def body(hp_ref, p_ref, w2_ref, o_ref):
        dinv = _dinv_col(hp_ref)
        ps = p_ref[0] + p_ref[1]
        even, odd = ps[:, :_H], ps[:, _H:]
        un = jnp.stack([even, odd], axis=1).reshape(_BT, _H)
        h2 = jnp.maximum(un * dinv, 0.0)
        o2 = _dot(h2, w2_ref[...]) * dinv
        o_ref[...] = _pack_cols(o2, _C)

    return pl.pallas_call(
        body,
        grid=(_GRID,),
        in_specs=[
            pl.BlockSpec((_NC, _BH, 128), lambda i: (0, i, 0)),
            pl.BlockSpec((_NC, _BP, 128), lambda i: (0, i, 0)),
            pl.BlockSpec((_H, _C), lambda i: (0, 0)),
        ],
        out_specs=pl.BlockSpec((_BH, 128), lambda i: (i, 0)),
        out_shape=jax.ShapeDtypeStruct((_NP * _C // 128, 128), jnp.float32),
    )(hpv, pv, w2)


def _tc_out(hpv, qv):
    def body(hp_ref, q_ref, o_ref):
        o_ref[...] = (q_ref[0] + q_ref[1]) * _dinv_packed(hp_ref)

    return pl.pallas_call(
        body,
        grid=(_GRID,),
        in_specs=[
            pl.BlockSpec((_NC, _BH, 128), lambda i: (0, i, 0)),
            pl.BlockSpec((_NC, _BH, 128), lambda i: (0, i, 0)),
        ],
        out_specs=pl.BlockSpec((_BH, 128), lambda i: (i, 0)),
        out_shape=jax.ShapeDtypeStruct((_NP * _C // 128, 128), jnp.float32),
    )(hpv, qv)


def kernel(x, edge_index, W1, W2):
    src = edge_index[0]
    src1 = _permute_idx(src, _H).reshape(_E // _K, _K)
    src2 = _permute_idx(src, _C).reshape(_E // _K, _K)
    dst = edge_index[1].reshape(_E // _K, _K)
    ones_k = jnp.ones((_K, _C), jnp.float32)
    zeros16 = jnp.zeros((_NP, _C), jnp.float32)
    zeros64 = jnp.zeros((_NP, _H), jnp.float32)

    hp = _make_hist()(ones_k, dst, zeros16)
    hpv = hp.reshape(_NC, _NP * _C // 128, 128)
    h1p = _tc_mm1(hpv, x, W1)
    h1v = h1p.reshape(_NP, _H)
    p = _make_spmm(_H)(h1v, src1, dst, zeros64)
    pv = p.reshape(_NC, _NP * _H // 128, 128)
    o2p = _tc_mm2(hpv, pv, W2)
    o2v = o2p.reshape(_NP, _C)
    q = _make_spmm(_C)(o2v, src2, dst, zeros16)
    qv = q.reshape(_NC, _NP * _C // 128, 128)
    return _tc_out(hpv, qv).reshape(_NP, _C)[:_N]
